# Initial kernel scaffold; baseline (speedup 1.0000x reference)
import jax
import jax.numpy as jnp
from jax.experimental import pallas as pl


def kernel(x, lower_indices, lower_values, upper_indices, upper_values, lower_W, lower_att, upper_W, upper_att, lin_W):
    raise NotImplementedError("write your pallas kernel here")


# speedup vs baseline: 4.4641x; 4.4641x over previous
import functools

import jax
import jax.numpy as jnp
from jax import lax
from jax.experimental import pallas as pl
from jax.experimental.pallas import tpu as pltpu
from jax.experimental.pallas import tpu_sc as plsc

N = 10000
E = 320000
D = 128
EPS = 1 + 1e-06

NC = 2
NS = 16
NP = 10240
NPART = NP // (NC * NS)
WIN = 1600
NWIN = E // WIN
BLK = 96
PCAP = BLK + 48


def _dense_body(x_ref, lw_ref, uw_ref, linw_ref, aml_ref, amu_ref,
                xml_ref, xmu_ref, wx_ref, avl_ref, avu_ref):
    xb = x_ref[...]
    ml = jnp.dot(xb, lw_ref[...], preferred_element_type=jnp.float32)
    mu = jnp.dot(xb, uw_ref[...], preferred_element_type=jnp.float32)
    xml_ref[...] = ml
    xmu_ref[...] = mu
    wx_ref[...] = jnp.dot(xb, linw_ref[...],
                          preferred_element_type=jnp.float32) * EPS
    avl_ref[...] = jnp.dot(ml, aml_ref[...], preferred_element_type=jnp.float32)
    avu_ref[...] = jnp.dot(mu, amu_ref[...], preferred_element_type=jnp.float32)


def _dense_stage(x, lower_W, upper_W, lin_W, am_l, am_u):
    blk = 1000
    grid = N // blk
    full = pl.BlockSpec((D, D), lambda i: (0, 0))
    row = pl.BlockSpec((blk, D), lambda i: (i, 0))
    return pl.pallas_call(
        _dense_body,
        grid=(grid,),
        in_specs=[row, full, full, full, full, full],
        out_specs=[row, row, row, row, row],
        out_shape=[jax.ShapeDtypeStruct((N, D), jnp.float32)] * 5,
    )(x, lower_W, upper_W, lin_W, am_l, am_u)


def _sc_edges(asl, atl, asu, atu, il, jl, vl, iu, ju, vu, xml, xmu):
    mesh = plsc.VectorSubcoreMesh(core_axis_name="c", subcore_axis_name="s",
                                  num_cores=NC, num_subcores=NS)

    @functools.partial(
        pl.kernel,
        out_type=[
            jax.ShapeDtypeStruct((NP, D), jnp.float32),
            jax.ShapeDtypeStruct((NP, 16), jnp.float32),
            jax.ShapeDtypeStruct((NP, D), jnp.float32),
            jax.ShapeDtypeStruct((NP, 16), jnp.float32),
        ],
        mesh=mesh,
        compiler_params=pltpu.CompilerParams(needs_layout_passes=False),
        scratch_types=[
            pltpu.VMEM((N,), jnp.float32),
            pltpu.VMEM((NPART,), jnp.float32),
            pltpu.VMEM((WIN,), jnp.int32),
            pltpu.VMEM((WIN,), jnp.int32),
            pltpu.VMEM((WIN,), jnp.float32),
            pltpu.VMEM((PCAP,), jnp.int32),
            pltpu.VMEM((PCAP,), jnp.int32),
            pltpu.VMEM((PCAP,), jnp.float32),
            pltpu.VMEM((BLK + 16,), jnp.int32),
            pltpu.VMEM((BLK + 16, D), jnp.float32),
            pltpu.VMEM((NPART + 1, D), jnp.float32),
            pltpu.VMEM((NPART + 1, 16), jnp.float32),
            pltpu.SMEM((1,), jnp.int32),
        ],
    )
    def k(asl_h, atl_h, asu_h, atu_h,
          il_h, jl_h, vl_h, iu_h, ju_h, vu_h,
          xml_h, xmu_h,
          nl_out, dl_out, nu_out, du_out,
          a_s, a_t_loc, ibuf, jbuf, vbuf,
          pend_i, pend_j, pend_v, gbuf, rowbuf, acc, dacc, cur_ref):
        c = lax.axis_index("c")
        s = lax.axis_index("s")
        lo = (c * NS + s) * NPART
        z16 = jnp.zeros((16,), jnp.float32)
        zi16 = jnp.zeros((16,), jnp.int32)

        for k2 in range(PCAP // 16):
            pend_i[pl.ds(k2 * 16, 16)] = zi16 + NPART
            pend_j[pl.ds(k2 * 16, 16)] = zi16
            pend_v[pl.ds(k2 * 16, 16)] = z16 - 1.0

        def flush(xm_h):
            for t in range(BLK // 16 + 1):
                gbuf[pl.ds(t * 16, 16)] = pend_j[pl.ds(t * 16, 16)]
            pltpu.sync_copy(xm_h.at[gbuf], rowbuf)

            def fl_body(t, carry):
                t16 = t * 16
                lv = pend_i[pl.ds(t16, 16)]
                pj = pend_j[pl.ds(t16, 16)]
                vv16 = pend_v[pl.ds(t16, 16)]
                sa = plsc.load_gather(a_s, [pj])
                ta = plsc.load_gather(a_t_loc, [jnp.minimum(lv, NPART - 1)])
                zz = sa + ta
                attn = jnp.where(zz > 0, zz, jnp.exp(zz) - 1.0)
                e16 = jnp.where(vv16 < 0, 0.0, jnp.exp(attn * vv16))
                for q in range(16):
                    r = t16 + q
                    lr = lv[q]
                    ev = jnp.full((16,), e16[q])
                    for f in range(D // 16):
                        sl = pl.ds(f * 16, 16)
                        acc[lr, sl] = acc[lr, sl] + rowbuf[r, sl] * ev
                    dacc[lr, :] = dacc[lr, :] + ev
                return carry
            lax.fori_loop(0, BLK // 16 + 1, fl_body, 0)

        branches = (
            (asl_h, atl_h, il_h, jl_h, vl_h, xml_h, nl_out, dl_out),
            (asu_h, atu_h, iu_h, ju_h, vu_h, xmu_h, nu_out, du_out),
        )
        for (as_h, at_h, i_h, j_h, v_h, xm_h, n_out, d_out) in branches:
            def zacc_body(q, carry):
                for f in range(D // 16):
                    acc[q, pl.ds(f * 16, 16)] = z16
                return carry
            lax.fori_loop(0, NPART + 1, zacc_body, 0)
            def zdacc_body(q, carry):
                dacc[q, :] = z16
                return carry
            lax.fori_loop(0, NPART + 1, zdacc_body, 0)
            cur_ref[0] = 0

            pltpu.sync_copy(as_h, a_s)
            pltpu.sync_copy(at_h.at[pl.ds(lo, NPART)], a_t_loc)


            def win_body(w, carry):
                wb = w * WIN
                pltpu.sync_copy(i_h.at[pl.ds(wb, WIN)], ibuf)
                pltpu.sync_copy(j_h.at[pl.ds(wb, WIN)], jbuf)
                pltpu.sync_copy(v_h.at[pl.ds(wb, WIN)], vbuf)

                def chunk_body(ch, c2):
                    off = ch * 16
                    iv = ibuf[pl.ds(off, 16)]
                    jv = jbuf[pl.ds(off, 16)]
                    lrow = iv - lo
                    m = (lrow >= 0) & (lrow < NPART)
                    cur = cur_ref[0]
                    key = jnp.where(m, lrow, jnp.int32(1 << 20))
                    ks, ps = plsc.sort_key_val(key, lax.iota(jnp.int32, 16))
                    cnt = jnp.sum(jnp.where(m, 1, 0))
                    lane = lax.iota(jnp.int32, 16)
                    live = lane < cnt
                    jv_c = plsc.load_gather(jbuf, [ps + off])
                    vv_c = jnp.where(live,
                                     plsc.load_gather(vbuf, [ps + off]), -1.0)
                    lrow_c = jnp.minimum(ks, NPART)
                    pend_i[pl.ds(cur, 16)] = lrow_c
                    pend_j[pl.ds(cur, 16)] = jv_c
                    pend_v[pl.ds(cur, 16)] = vv_c
                    newcur = cur + cnt

                    @pl.when(newcur >= BLK)
                    def _():
                        flush(xm_h)
                        for t in range(BLK // 16 + 1):
                            pend_i[pl.ds(t * 16, 16)] = zi16 + NPART
                            pend_v[pl.ds(t * 16, 16)] = z16 - 1.0
                        cur_ref[0] = 0

                    @pl.when(newcur < BLK)
                    def _():
                        cur_ref[0] = newcur
                    return c2
                lax.fori_loop(0, WIN // 16, chunk_body, 0)
                return carry
            lax.fori_loop(0, NWIN, win_body, 0)

            cur = cur_ref[0]
            for t in range(BLK // 16 + 1):
                lidx = lax.iota(jnp.int32, 16) + (t * 16)
                mpad = lidx >= cur
                wv = pend_v[pl.ds(t * 16, 16)]
                wi = pend_i[pl.ds(t * 16, 16)]
                pend_v[pl.ds(t * 16, 16)] = jnp.where(mpad, -1.0, wv)
                pend_i[pl.ds(t * 16, 16)] = jnp.where(mpad, NPART, wi)

            @pl.when(cur > 0)
            def _():
                flush(xm_h)

            pltpu.sync_copy(acc.at[pl.ds(0, NPART)],
                            n_out.at[pl.ds(lo, NPART)])
            pltpu.sync_copy(dacc.at[pl.ds(0, NPART)],
                            d_out.at[pl.ds(lo, NPART)])

    return k(asl, atl, asu, atu, il, jl, vl, iu, ju, vu, xml, xmu)


def _combine_body(nl_ref, dl_ref, nu_ref, du_ref, wx_ref, o_ref):
    rl = nl_ref[...] * jnp.where(dl_ref[...] != 0, 1.0 / dl_ref[...], 0.0)
    ru = nu_ref[...] * jnp.where(du_ref[...] != 0, 1.0 / du_ref[...], 0.0)
    o_ref[...] = jnp.maximum(rl + ru + wx_ref[...], 0.0)


def _combine(nl, dl, nu, du, wx):
    blk = 1000
    grid = N // blk
    row = pl.BlockSpec((blk, D), lambda i: (i, 0))
    return pl.pallas_call(
        _combine_body,
        grid=(grid,),
        in_specs=[row, row, row, row, row],
        out_specs=row,
        out_shape=jax.ShapeDtypeStruct((N, D), jnp.float32),
    )(nl, dl, nu, du, wx)


def kernel(x, lower_indices, lower_values, upper_indices, upper_values,
           lower_W, lower_att, upper_W, upper_att, lin_W):
    am_l = jnp.zeros((D, D), jnp.float32)
    am_l = am_l.at[:, 0].set(lower_att[:D]).at[:, 1].set(lower_att[D:])
    am_u = jnp.zeros((D, D), jnp.float32)
    am_u = am_u.at[:, 0].set(upper_att[:D]).at[:, 1].set(upper_att[D:])

    xml, xmu, wx, avl, avu = _dense_stage(x, lower_W, upper_W, lin_W,
                                          am_l, am_u)

    nl, dl, nu, du = _sc_edges(
        avl[:, 0], avl[:, 1], avu[:, 0], avu[:, 1],
        lower_indices[0], lower_indices[1], lower_values,
        upper_indices[0], upper_indices[1], upper_values,
        xml, xmu)

    dlb = jnp.broadcast_to(dl[:N, 0:1], (N, D))
    dub = jnp.broadcast_to(du[:N, 0:1], (N, D))
    return _combine(nl[:N, :], dlb, nu[:N, :], dub, wx)
